# baseline (speedup 1.0000x reference)
import functools

import jax
import jax.numpy as jnp
from jax import lax
from jax.experimental import pallas as pl
from jax.experimental.pallas import tpu as pltpu
from jax.experimental.pallas import tpu_sc as plsc

N_NODES = 10000
M_NBR = 32
ATOM = 128
NBR = 16
WIDTH = 2 * ATOM + 2 * NBR
EDGES = N_NODES * M_NBR

TILE_N = 200
GRID_N = N_NODES // TILE_N
TILE_E = TILE_N * M_NBR

PQ_TILE = 2000

NW = 32
E_PER_W = EDGES // NW
CHUNK = 40
NBUF = 10
N_GROUPS = E_PER_W // (CHUNK * NBUF)
EPS = 1e-5


def _softplus(x):
    return jnp.logaddexp(0.0, x)


def _pq_body(atom_ref, wp_ref, p_ref):
    p_ref[...] = jnp.dot(
        atom_ref[...], wp_ref[...], preferred_element_type=jnp.float32
    )


def _pq(atom, wpT):
    grid = N_NODES // PQ_TILE
    return pl.pallas_call(
        _pq_body,
        grid=(grid,),
        in_specs=[
            pl.BlockSpec((PQ_TILE, ATOM), lambda i: (i, 0)),
            pl.BlockSpec((ATOM, WIDTH), lambda i: (0, 0)),
        ],
        out_specs=pl.BlockSpec((PQ_TILE, WIDTH), lambda i: (i, 0)),
        out_shape=jax.ShapeDtypeStruct((N_NODES, WIDTH), jnp.float32),
    )(atom, wpT)


def _sc_gather_body(tab_hbm, idx_hbm, out_hbm, idx_all, bufs, sem_g, sem_s):
    wid = lax.axis_index("s") * 2 + lax.axis_index("c")
    base = wid * E_PER_W
    pltpu.sync_copy(idx_hbm.at[pl.ds(base, E_PER_W)], idx_all)

    def gather(c, b):
        return pltpu.make_async_copy(
            tab_hbm.at[idx_all.at[pl.ds(c * CHUNK, CHUNK)]], bufs.at[b], sem_g
        )

    def store(c, b):
        return pltpu.make_async_copy(
            bufs.at[b], out_hbm.at[pl.ds(base + c * CHUNK, CHUNK)], sem_s
        )

    def group(g, carry):
        @pl.when(g > 0)
        def _():
            for b in range(NBUF):
                store(0, b).wait()
        for b in range(NBUF):
            gather(g * NBUF + b, b).start()
        for b in range(NBUF):
            gather(g * NBUF + b, b).wait()
            store(g * NBUF + b, b).start()
        return carry

    lax.fori_loop(0, N_GROUPS, group, 0)
    for b in range(NBUF):
        store(0, b).wait()


def _sc_gather(atom, idx_flat):
    mesh = plsc.VectorSubcoreMesh(core_axis_name="c", subcore_axis_name="s")
    fn = functools.partial(
        pl.kernel,
        mesh=mesh,
        out_type=jax.ShapeDtypeStruct((EDGES, ATOM), jnp.float32),
        scratch_types=[
            pltpu.VMEM((E_PER_W,), jnp.int32),
            pltpu.VMEM((NBUF, CHUNK, ATOM), jnp.float32),
            pltpu.SemaphoreType.DMA,
            pltpu.SemaphoreType.DMA,
        ],
    )(_sc_gather_body)
    return fn(atom, idx_flat)


def _edge_y(p_ref, ag_ref, nbr_ref, wq_ref, wc_ref):
    qg = jnp.dot(ag_ref[...], wq_ref[...], preferred_element_type=jnp.float32)
    r = jnp.dot(
        nbr_ref[...].reshape(TILE_E, NBR), wc_ref[...],
        preferred_element_type=jnp.float32,
    )
    pb = jnp.broadcast_to(
        p_ref[...][:, None, :], (TILE_N, M_NBR, WIDTH)
    ).reshape(TILE_E, WIDTH)
    return pb + qg + r


def _stats_body(p_ref, ag_ref, nbr_ref, wq_ref, wc_ref, s_ref, ss_ref):
    y = _edge_y(p_ref, ag_ref, nbr_ref, wq_ref, wc_ref)

    @pl.when(pl.program_id(0) == 0)
    def _():
        s_ref[...] = jnp.zeros_like(s_ref)
        ss_ref[...] = jnp.zeros_like(ss_ref)

    s_ref[...] += jnp.sum(y, axis=0, keepdims=True)
    ss_ref[...] += jnp.sum(y * y, axis=0, keepdims=True)


def _stats():
    return pl.pallas_call(
        _stats_body,
        grid=(GRID_N,),
        in_specs=[
            pl.BlockSpec((TILE_N, WIDTH), lambda i: (i, 0)),
            pl.BlockSpec((TILE_E, ATOM), lambda i: (i, 0)),
            pl.BlockSpec((TILE_N, M_NBR, NBR), lambda i: (i, 0, 0)),
            pl.BlockSpec((ATOM, WIDTH), lambda i: (0, 0)),
            pl.BlockSpec((NBR, WIDTH), lambda i: (0, 0)),
        ],
        out_specs=[
            pl.BlockSpec((1, WIDTH), lambda i: (0, 0)),
            pl.BlockSpec((1, WIDTH), lambda i: (0, 0)),
        ],
        out_shape=[
            jax.ShapeDtypeStruct((1, WIDTH), jnp.float32),
            jax.ShapeDtypeStruct((1, WIDTH), jnp.float32),
        ],
    )


def _apply_body(p_ref, ag_ref, nbr_ref, wq_ref, wc_ref, scale_ref, shift_ref,
                asum_ref, out2_ref, s2_ref, ss2_ref):
    y = _edge_y(p_ref, ag_ref, nbr_ref, wq_ref, wc_ref)
    z = y * scale_ref[...] + shift_ref[...]

    af = jax.nn.sigmoid(z[:, :ATOM])
    ac = _softplus(z[:, ATOM:2 * ATOM])
    asum = jnp.sum((af * ac).reshape(TILE_N, M_NBR, ATOM), axis=1)
    asum_ref[...] = asum

    @pl.when(pl.program_id(0) == 0)
    def _():
        s2_ref[...] = jnp.zeros_like(s2_ref)
        ss2_ref[...] = jnp.zeros_like(ss2_ref)

    s2_ref[...] += jnp.sum(asum, axis=0, keepdims=True)
    ss2_ref[...] += jnp.sum(asum * asum, axis=0, keepdims=True)

    zbt = z[:, 2 * ATOM:].T
    gt = jax.nn.sigmoid(zbt[:NBR, :]) * _softplus(zbt[NBR:, :])
    nbrt = nbr_ref[...].reshape(TILE_E, NBR).T
    out2_ref[...] = _softplus(nbrt + gt).T.reshape(TILE_N, M_NBR, NBR)


def _apply():
    return pl.pallas_call(
        _apply_body,
        grid=(GRID_N,),
        in_specs=[
            pl.BlockSpec((TILE_N, WIDTH), lambda i: (i, 0)),
            pl.BlockSpec((TILE_E, ATOM), lambda i: (i, 0)),
            pl.BlockSpec((TILE_N, M_NBR, NBR), lambda i: (i, 0, 0)),
            pl.BlockSpec((ATOM, WIDTH), lambda i: (0, 0)),
            pl.BlockSpec((NBR, WIDTH), lambda i: (0, 0)),
            pl.BlockSpec((1, WIDTH), lambda i: (0, 0)),
            pl.BlockSpec((1, WIDTH), lambda i: (0, 0)),
        ],
        out_specs=[
            pl.BlockSpec((TILE_N, ATOM), lambda i: (i, 0)),
            pl.BlockSpec((TILE_N, M_NBR, NBR), lambda i: (i, 0, 0)),
            pl.BlockSpec((1, ATOM), lambda i: (0, 0)),
            pl.BlockSpec((1, ATOM), lambda i: (0, 0)),
        ],
        out_shape=[
            jax.ShapeDtypeStruct((N_NODES, ATOM), jnp.float32),
            jax.ShapeDtypeStruct((N_NODES, M_NBR, NBR), jnp.float32),
            jax.ShapeDtypeStruct((1, ATOM), jnp.float32),
            jax.ShapeDtypeStruct((1, ATOM), jnp.float32),
        ],
    )


def _out1_body(atom_ref, asum_ref, sc2_ref, sh2_ref, out_ref):
    out_ref[...] = _softplus(
        atom_ref[...] + asum_ref[...] * sc2_ref[...] + sh2_ref[...]
    )


def _out1(atom, asum, scale2, shift2):
    grid = N_NODES // PQ_TILE
    return pl.pallas_call(
        _out1_body,
        grid=(grid,),
        in_specs=[
            pl.BlockSpec((PQ_TILE, ATOM), lambda i: (i, 0)),
            pl.BlockSpec((PQ_TILE, ATOM), lambda i: (i, 0)),
            pl.BlockSpec((1, ATOM), lambda i: (0, 0)),
            pl.BlockSpec((1, ATOM), lambda i: (0, 0)),
        ],
        out_specs=pl.BlockSpec((PQ_TILE, ATOM), lambda i: (i, 0)),
        out_shape=jax.ShapeDtypeStruct((N_NODES, ATOM), jnp.float32),
    )(atom, asum, scale2, shift2)


def kernel(atom_in_fea, nbr_fea, nbr_fea_idx, W1, W2, g1a, b1a, g1b, b1b, g2a, b2a):
    wcat = jnp.concatenate([W1, W2], axis=0)
    wpT = wcat[:, :ATOM].T
    wqT = wcat[:, ATOM:2 * ATOM].T
    wcT = wcat[:, 2 * ATOM:].T
    gamma = jnp.concatenate([g1a, g1b])[None, :]
    beta = jnp.concatenate([b1a, b1b])[None, :]

    p = _pq(atom_in_fea, wpT)
    ag = _sc_gather(atom_in_fea, nbr_fea_idx.reshape(-1))

    s, ss = _stats()(p, ag, nbr_fea, wqT, wcT)
    mean = s / EDGES
    var = ss / EDGES - mean * mean
    scale = gamma * lax.rsqrt(var + EPS)
    shift = beta - mean * scale

    asum, out2, s2, ss2 = _apply()(p, ag, nbr_fea, wqT, wcT, scale, shift)
    mean2 = s2 / N_NODES
    var2 = ss2 / N_NODES - mean2 * mean2
    scale2 = g2a[None, :] * lax.rsqrt(var2 + EPS)
    shift2 = b2a[None, :] - mean2 * scale2

    out1 = _out1(atom_in_fea, asum, scale2, shift2)
    return (out1, out2)
